# speedup vs baseline: 2832.5111x; 2832.5111x over previous
import jax
import jax.numpy as jnp
from jax import lax
from jax.experimental import pallas as pl

_IMG = 512.0
_MAX_OUT = 100
_IOU_T = 0.5
_SCORE_T = 0.5
_MIN_AREA = 100.0
_N = 20000
_ROWS = 160
_COLS = 128
_NPAD = _ROWS * _COLS


def _nms_body(y1_ref, x1_ref, y2_ref, x2_ref, sc_ref, out_ref):
    y1 = y1_ref[:]
    x1 = x1_ref[:]
    y2 = y2_ref[:]
    x2 = x2_ref[:]
    scores = sc_ref[:]
    h = y2 - y1
    w = x2 - x1
    area = h * w
    valid = (
        (w / _IMG <= 1.0)
        & (h / _IMG <= 1.0)
        & (area > _MIN_AREA)
        & (scores >= _SCORE_T)
    )
    s0 = jnp.where(valid, scores, -1.0)
    rows = lax.broadcasted_iota(jnp.int32, (_ROWS, _COLS), 0)
    cols = lax.broadcasted_iota(jnp.int32, (_ROWS, _COLS), 1)
    lin = rows * _COLS + cols
    a2 = (y2 - y1) * (x2 - x1)
    ci = lax.broadcasted_iota(jnp.int32, (1, _COLS), 1)

    def pick(sarr):
        idx = jnp.min(jnp.where(sarr == jnp.max(sarr), lin, jnp.int32(2**30)))
        eq = lin == idx
        gy1 = jnp.sum(jnp.where(eq, y1, 0.0))
        gx1 = jnp.sum(jnp.where(eq, x1, 0.0))
        gy2 = jnp.sum(jnp.where(eq, y2, 0.0))
        gx2 = jnp.sum(jnp.where(eq, x2, 0.0))
        return eq, gy1, gx1, gy2, gx2

    def body(t, carry):
        s_act, s_fill = carry
        m = jnp.max(s_act)

        def keeper(args):
            s_act, s_fill = args
            eq, gy1, gx1, gy2, gx2 = pick(s_act)
            iy1 = jnp.maximum(gy1, y1)
            ix1 = jnp.maximum(gx1, x1)
            iy2 = jnp.minimum(gy2, y2)
            ix2 = jnp.minimum(gx2, x2)
            inter = jnp.maximum(iy2 - iy1, 0.0) * jnp.maximum(ix2 - ix1, 0.0)
            a1 = (gy2 - gy1) * (gx2 - gx1)
            union = a1 + a2 - inter
            iou = inter / jnp.maximum(union, 1e-8)
            sup = iou > _IOU_T
            return (
                jnp.where(sup, -3.0, s_act),
                jnp.where(eq, -3.0, s_fill),
                gy1,
                gx1,
                gy2,
                gx2,
                m,
            )

        def filler(args):
            s_act, s_fill = args
            eq, gy1, gx1, gy2, gx2 = pick(s_fill)
            return (s_act, jnp.where(eq, -3.0, s_fill), gy1, gx1, gy2, gx2, 0.0)

        s_act, s_fill, gy1, gx1, gy2, gx2, sc_out = lax.cond(
            m > 0.0, keeper, filler, (s_act, s_fill)
        )
        cy1 = jnp.clip(gy1, 0.0, _IMG)
        cx1 = jnp.clip(gx1, 0.0, _IMG)
        cy2 = jnp.clip(gy2, 0.0, _IMG)
        cx2 = jnp.clip(gx2, 0.0, _IMG)
        rowv = (
            jnp.where(ci == 0, cy1, 0.0)
            + jnp.where(ci == 1, cx1, 0.0)
            + jnp.where(ci == 2, cy2, 0.0)
            + jnp.where(ci == 3, cx2, 0.0)
            + jnp.where(ci == 4, sc_out, 0.0)
        )
        out_ref[pl.ds(t, 1), :] = rowv
        return s_act, s_fill

    lax.fori_loop(0, _MAX_OUT, body, (s0, s0))


def _run_nms(y1, x1, y2, x2, s):
    return pl.pallas_call(
        _nms_body,
        out_shape=jax.ShapeDtypeStruct((_MAX_OUT, _COLS), jnp.float32),
    )(y1, x1, y2, x2, s)


@jax.jit
def kernel(boxes, scores):
    pad = _NPAD - _N
    b = jnp.pad(boxes, ((0, pad), (0, 0)))
    s = jnp.pad(scores, ((0, pad),)).reshape(_ROWS, _COLS)
    y1 = b[:, 0].reshape(_ROWS, _COLS)
    x1 = b[:, 1].reshape(_ROWS, _COLS)
    y2 = b[:, 2].reshape(_ROWS, _COLS)
    x2 = b[:, 3].reshape(_ROWS, _COLS)
    out = _run_nms(y1, x1, y2, x2, s)
    return out[:, :5]
